# Initial kernel scaffold; baseline (speedup 1.0000x reference)
import jax
import jax.numpy as jnp
from jax.experimental import pallas as pl


def kernel(ent_embs, rels, neg_idx, rel_emb_weight):
    raise NotImplementedError("write your pallas kernel here")


# speedup vs baseline: 5.9999x; 5.9999x over previous
import functools

import jax
import jax.numpy as jnp
from jax import lax
from jax.experimental import pallas as pl
from jax.experimental.pallas import tpu as pltpu
from jax.experimental.pallas import tpu_sc as plsc

B = 4096
DIM = 128
NNEG = 64
REG = 0.01
HALF = DIM // 2
NCH = HALF // 16

_info = plsc.get_sparse_core_info()
NC, NS, L = _info.num_cores, _info.num_subcores, _info.num_lanes
NW = NC * NS
BPW = B // NW
GSZ = 2 * NNEG


def _sc_scores(ent_flat, rel_idx, neg_idx_flat, rel_tab):
  mesh = plsc.VectorSubcoreMesh(core_axis_name="c", subcore_axis_name="s")

  @functools.partial(
      pl.kernel,
      out_type=[
          jax.ShapeDtypeStruct((B,), jnp.float32),
          jax.ShapeDtypeStruct((B, NNEG), jnp.float32),
          jax.ShapeDtypeStruct((NW, 4, L), jnp.float32),
      ],
      mesh=mesh,
      compiler_params=pltpu.CompilerParams(needs_layout_passes=False),
      scratch_types=[
          pltpu.VMEM((BPW * GSZ,), jnp.int32),
          pltpu.VMEM((BPW,), jnp.int32),
          pltpu.VMEM((BPW, DIM), jnp.float32),
          pltpu.VMEM((2 * BPW, DIM), jnp.float32),
          pltpu.VMEM((2, GSZ, DIM), jnp.float32),
          pltpu.VMEM((BPW, NNEG), jnp.float32),
          pltpu.VMEM((BPW,), jnp.float32),
          pltpu.VMEM((4, L), jnp.float32),
          pltpu.SemaphoreType.DMA,
          pltpu.SemaphoreType.DMA,
          pltpu.SemaphoreType.DMA,
      ],
  )
  def sck(ent_hbm, relidx_hbm, negidx_hbm, reltab_hbm,
          pos_out, neg_out, sq_out,
          negidx_v, relidx_v, rele_v, posrows_v, rows_v, scores_v, pos_v,
          sq_v, sem0, sem1, semg):
    wid = lax.axis_index("s") * NC + lax.axis_index("c")
    base = wid * BPW
    lanes = lax.iota(jnp.int32, L)
    zero = jnp.zeros((L,), jnp.float32)

    pltpu.sync_copy(negidx_hbm.at[pl.ds(base * GSZ, BPW * GSZ)], negidx_v)
    pltpu.sync_copy(relidx_hbm.at[pl.ds(base, BPW)], relidx_v)
    pltpu.sync_copy(ent_hbm.at[pl.ds(base * 2, BPW * 2)], posrows_v)
    pltpu.async_copy(reltab_hbm.at[relidx_v], rele_v, semg).wait()

    pltpu.async_copy(ent_hbm.at[negidx_v.at[pl.ds(0, GSZ)]],
                     rows_v.at[0], sem0)
    pltpu.async_copy(ent_hbm.at[negidx_v.at[pl.ds(GSZ, GSZ)]],
                     rows_v.at[1], sem1)

    def pos_body(bg, carry):
      sqh, sqt, sqr = carry
      pvec = zero
      for bb in range(L):
        b = bg * L + bb
        acc = zero
        for c in range(NCH):
          lo = c * 16
          hi = HALF + c * 16
          rr = rele_v[b, pl.ds(lo, 16)]
          ri = rele_v[b, pl.ds(hi, 16)]
          hre = posrows_v[2 * b, pl.ds(lo, 16)]
          him = posrows_v[2 * b, pl.ds(hi, 16)]
          tre = posrows_v[2 * b + 1, pl.ds(lo, 16)]
          tim = posrows_v[2 * b + 1, pl.ds(hi, 16)]
          acc = acc + rr * (hre * tre + him * tim) \
                    + ri * (hre * tim - him * tre)
          sqh = sqh + hre * hre + him * him
          sqt = sqt + tre * tre + tim * tim
          sqr = sqr + rr * rr + ri * ri
        pvec = jnp.where(lanes == bb, jnp.sum(acc), pvec)
      pos_v[pl.ds(bg * L, L)] = pvec
      return sqh, sqt, sqr

    sqh, sqt, sqr = lax.fori_loop(0, BPW // L, pos_body, (zero, zero, zero))
    sq_v[0, :] = sqh
    sq_v[1, :] = sqt
    sq_v[2, :] = sqr
    sq_v[3, :] = zero

    def neg_body(i, _):
      for q, sem in ((0, sem0), (1, sem1)):
        b = i * 2 + q
        pltpu.make_async_copy(
            ent_hbm.at[negidx_v.at[pl.ds(b * GSZ, GSZ)]],
            rows_v.at[q], sem).wait()

        rr = [rele_v[b, pl.ds(c * 16, 16)] for c in range(NCH)]
        ri = [rele_v[b, pl.ds(HALF + c * 16, 16)] for c in range(NCH)]

        def grp_body(g, _, q=q, rr=rr, ri=ri, b=b):
          svec = zero
          for jj in range(L):
            j = g * L + jj
            acc = zero
            for c in range(NCH):
              lo = c * 16
              hi = HALF + c * 16
              hre = rows_v[q, 2 * j, pl.ds(lo, 16)]
              him = rows_v[q, 2 * j, pl.ds(hi, 16)]
              tre = rows_v[q, 2 * j + 1, pl.ds(lo, 16)]
              tim = rows_v[q, 2 * j + 1, pl.ds(hi, 16)]
              acc = acc + rr[c] * (hre * tre + him * tim) \
                        + ri[c] * (hre * tim - him * tre)
            svec = jnp.where(lanes == jj, jnp.sum(acc), svec)
          scores_v[b, pl.ds(g * L, L)] = svec
          return 0

        lax.fori_loop(0, NNEG // L, grp_body, 0)

        nxt = b + 2

        @pl.when(nxt < BPW)
        def _():
          pltpu.async_copy(
              ent_hbm.at[negidx_v.at[pl.ds(nxt * GSZ, GSZ)]],
              rows_v.at[q], sem)
      return 0

    lax.fori_loop(0, BPW // 2, neg_body, 0)

    pltpu.sync_copy(pos_v, pos_out.at[pl.ds(base, BPW)])
    pltpu.sync_copy(scores_v, neg_out.at[pl.ds(base, BPW)])
    pltpu.sync_copy(sq_v, sq_out.at[wid])

  return sck(ent_flat, rel_idx, neg_idx_flat, rel_tab)


def _tc_loss(pos2d, neg2d, sq2d):
  def body(pos_ref, neg_ref, sq_ref, out_ref):
    pos = pos_ref[...]
    neg = neg_ref[...]
    sq = sq_ref[...]

    def sp(x):
      return jnp.maximum(x, 0.0) + jnp.log1p(jnp.exp(-jnp.abs(x)))

    model = 0.5 * (jnp.sum(sp(-pos)) / B + jnp.sum(sp(neg)) / (B * NNEG))
    reg = REG * jnp.sum(sq) / (3.0 * B * DIM)
    out_ref[0, 0] = model + reg

  out = pl.pallas_call(
      body,
      out_shape=jax.ShapeDtypeStruct((1, 1), jnp.float32),
      out_specs=pl.BlockSpec(memory_space=pltpu.SMEM),
  )(pos2d, neg2d, sq2d)
  return out[0, 0]


def kernel(ent_embs, rels, neg_idx, rel_emb_weight):
  ent_flat = ent_embs.reshape(B * 2, DIM)
  rel_idx = rels.reshape(B).astype(jnp.int32)
  neg_flat = neg_idx.reshape(B * GSZ).astype(jnp.int32)
  pos, neg, sq = _sc_scores(ent_flat, rel_idx, neg_flat, rel_emb_weight)
  return _tc_loss(pos.reshape(NW, BPW), neg, sq.reshape(L, DIM))


# speedup vs baseline: 6.3497x; 1.0583x over previous
import functools

import jax
import jax.numpy as jnp
from jax import lax
from jax.experimental import pallas as pl
from jax.experimental.pallas import tpu as pltpu
from jax.experimental.pallas import tpu_sc as plsc

B = 4096
DIM = 128
NNEG = 64
REG = 0.01
HALF = DIM // 2
NCH = HALF // 16

_info = plsc.get_sparse_core_info()
NC, NS, L = _info.num_cores, _info.num_subcores, _info.num_lanes
NW = NC * NS
BPW = B // NW
GSZ = 2 * NNEG
NBUF = 4


def _sc_scores(ent_flat, rel_idx, neg_idx_flat, rel_tab):
  mesh = plsc.VectorSubcoreMesh(core_axis_name="c", subcore_axis_name="s")

  @functools.partial(
      pl.kernel,
      out_type=[
          jax.ShapeDtypeStruct((B,), jnp.float32),
          jax.ShapeDtypeStruct((B, NNEG), jnp.float32),
          jax.ShapeDtypeStruct((NW, 4, L), jnp.float32),
      ],
      mesh=mesh,
      compiler_params=pltpu.CompilerParams(needs_layout_passes=False),
      scratch_types=[
          pltpu.VMEM((BPW * GSZ,), jnp.int32),
          pltpu.VMEM((BPW,), jnp.int32),
          pltpu.VMEM((BPW, DIM), jnp.float32),
          pltpu.VMEM((NBUF, GSZ, DIM), jnp.float32),
          pltpu.VMEM((BPW, NNEG), jnp.float32),
          pltpu.VMEM((BPW,), jnp.float32),
          pltpu.VMEM((4, L), jnp.float32),
          pltpu.VMEM((L, L), jnp.float32),
          pltpu.SemaphoreType.DMA,
          pltpu.SemaphoreType.DMA,
          pltpu.SemaphoreType.DMA,
          pltpu.SemaphoreType.DMA,
          pltpu.SemaphoreType.DMA,
      ],
  )
  def sck(ent_hbm, relidx_hbm, negidx_hbm, reltab_hbm,
          pos_out, neg_out, sq_out,
          negidx_v, relidx_v, rele_v, rows_v, scores_v, pos_v,
          sq_v, accv, sem0, sem1, sem2, sem3, semg):
    wid = lax.axis_index("s") * NC + lax.axis_index("c")
    base = wid * BPW
    lanes = lax.iota(jnp.int32, L)
    zero = jnp.zeros((L,), jnp.float32)
    cols = [jnp.full((L,), c, jnp.int32) for c in range(L)]
    sems = (sem0, sem1, sem2, sem3)

    pltpu.sync_copy(negidx_hbm.at[pl.ds(base * GSZ, BPW * GSZ)], negidx_v)
    pltpu.sync_copy(relidx_hbm.at[pl.ds(base, BPW)], relidx_v)
    pltpu.sync_copy(ent_hbm.at[pl.ds(base * 2, GSZ)], rows_v.at[2])
    pltpu.sync_copy(ent_hbm.at[pl.ds(base * 2 + GSZ, GSZ)], rows_v.at[3])
    pltpu.async_copy(reltab_hbm.at[relidx_v], rele_v, semg).wait()

    pltpu.async_copy(ent_hbm.at[negidx_v.at[pl.ds(0, GSZ)]],
                     rows_v.at[0], sem0)
    pltpu.async_copy(ent_hbm.at[negidx_v.at[pl.ds(GSZ, GSZ)]],
                     rows_v.at[1], sem1)

    def colsum():
      svec = load_col(0)
      for c in range(1, L):
        svec = svec + load_col(c)
      return svec

    def load_col(c):
      return plsc.load_gather(accv, [lanes, cols[c]])

    carry = (zero, zero, zero)
    for half in range(2):

      def pos_body(g, carry, half=half):
        sqh, sqt, sqr = carry
        for jj in range(L):
          bl = g * L + jj
          b = half * 64 + bl
          acc = zero
          for c in range(NCH):
            lo = c * 16
            hi = HALF + c * 16
            rr = rele_v[b, pl.ds(lo, 16)]
            ri = rele_v[b, pl.ds(hi, 16)]
            hre = rows_v[2 + half, 2 * bl, pl.ds(lo, 16)]
            him = rows_v[2 + half, 2 * bl, pl.ds(hi, 16)]
            tre = rows_v[2 + half, 2 * bl + 1, pl.ds(lo, 16)]
            tim = rows_v[2 + half, 2 * bl + 1, pl.ds(hi, 16)]
            acc = acc + rr * (hre * tre + him * tim) \
                      + ri * (hre * tim - him * tre)
            sqh = sqh + hre * hre + him * him
            sqt = sqt + tre * tre + tim * tim
            sqr = sqr + rr * rr + ri * ri
          accv[jj, :] = acc
        pos_v[pl.ds(half * 64 + g * L, L)] = colsum()
        return sqh, sqt, sqr

      carry = lax.fori_loop(0, 64 // L, pos_body, carry)

    sqh, sqt, sqr = carry
    sq_v[0, :] = sqh
    sq_v[1, :] = sqt
    sq_v[2, :] = sqr
    sq_v[3, :] = zero

    pltpu.async_copy(ent_hbm.at[negidx_v.at[pl.ds(2 * GSZ, GSZ)]],
                     rows_v.at[2], sem2)
    pltpu.async_copy(ent_hbm.at[negidx_v.at[pl.ds(3 * GSZ, GSZ)]],
                     rows_v.at[3], sem3)

    def neg_body(i, _):
      for q in range(NBUF):
        b = i * NBUF + q
        sem = sems[q]
        pltpu.make_async_copy(
            ent_hbm.at[negidx_v.at[pl.ds(b * GSZ, GSZ)]],
            rows_v.at[q], sem).wait()

        rr = [rele_v[b, pl.ds(c * 16, 16)] for c in range(NCH)]
        ri = [rele_v[b, pl.ds(HALF + c * 16, 16)] for c in range(NCH)]

        def grp_body(g, _, q=q, rr=rr, ri=ri, b=b):
          for jj in range(L):
            j = g * L + jj
            acc = zero
            for c in range(NCH):
              lo = c * 16
              hi = HALF + c * 16
              hre = rows_v[q, 2 * j, pl.ds(lo, 16)]
              him = rows_v[q, 2 * j, pl.ds(hi, 16)]
              tre = rows_v[q, 2 * j + 1, pl.ds(lo, 16)]
              tim = rows_v[q, 2 * j + 1, pl.ds(hi, 16)]
              acc = acc + rr[c] * (hre * tre + him * tim) \
                        + ri[c] * (hre * tim - him * tre)
            accv[jj, :] = acc
          scores_v[b, pl.ds(g * L, L)] = colsum()
          return 0

        lax.fori_loop(0, NNEG // L, grp_body, 0)

        nxt = b + NBUF

        @pl.when(nxt < BPW)
        def _():
          pltpu.async_copy(
              ent_hbm.at[negidx_v.at[pl.ds(nxt * GSZ, GSZ)]],
              rows_v.at[q], sem)
      return 0

    lax.fori_loop(0, BPW // NBUF, neg_body, 0)

    pltpu.sync_copy(pos_v, pos_out.at[pl.ds(base, BPW)])
    pltpu.sync_copy(scores_v, neg_out.at[pl.ds(base, BPW)])
    pltpu.sync_copy(sq_v, sq_out.at[wid])

  return sck(ent_flat, rel_idx, neg_idx_flat, rel_tab)


def _tc_loss(pos2d, neg2d, sq2d):
  def body(pos_ref, neg_ref, sq_ref, out_ref):
    pos = pos_ref[...]
    neg = neg_ref[...]
    sq = sq_ref[...]

    def sp(x):
      return jnp.maximum(x, 0.0) + jnp.log1p(jnp.exp(-jnp.abs(x)))

    model = 0.5 * (jnp.sum(sp(-pos)) / B + jnp.sum(sp(neg)) / (B * NNEG))
    reg = REG * jnp.sum(sq) / (3.0 * B * DIM)
    out_ref[0, 0] = model + reg

  out = pl.pallas_call(
      body,
      out_shape=jax.ShapeDtypeStruct((1, 1), jnp.float32),
      out_specs=pl.BlockSpec(memory_space=pltpu.SMEM),
  )(pos2d, neg2d, sq2d)
  return out[0, 0]


def kernel(ent_embs, rels, neg_idx, rel_emb_weight):
  ent_flat = ent_embs.reshape(B * 2, DIM)
  rel_idx = rels.reshape(B).astype(jnp.int32)
  neg_flat = neg_idx.reshape(B * GSZ).astype(jnp.int32)
  pos, neg, sq = _sc_scores(ent_flat, rel_idx, neg_flat, rel_emb_weight)
  return _tc_loss(pos.reshape(NW, BPW), neg, sq.reshape(L, DIM))


# speedup vs baseline: 9.5490x; 1.5039x over previous
import functools

import jax
import jax.numpy as jnp
from jax import lax
from jax.experimental import pallas as pl
from jax.experimental.pallas import tpu as pltpu
from jax.experimental.pallas import tpu_sc as plsc

B = 4096
DIM = 128
NNEG = 64
REG = 0.01
HALF = DIM // 2
NCH = HALF // 16

_info = plsc.get_sparse_core_info()
NC, NS, L = _info.num_cores, _info.num_subcores, _info.num_lanes
NW = NC * NS
BPW = B // NW
GSZ = 2 * NNEG
NBUF = 3


def _sc_scores(ent3, rel_idx, hidx, tidx, rel_tab):
  mesh = plsc.VectorSubcoreMesh(core_axis_name="c", subcore_axis_name="s")

  @functools.partial(
      pl.kernel,
      out_type=[
          jax.ShapeDtypeStruct((NW, BPW), jnp.float32),
          jax.ShapeDtypeStruct((B, NNEG), jnp.float32),
          jax.ShapeDtypeStruct((NW, 4, L), jnp.float32),
      ],
      mesh=mesh,
      compiler_params=pltpu.CompilerParams(needs_layout_passes=False),
      scratch_types=[
          pltpu.VMEM((BPW, NNEG), jnp.int32),
          pltpu.VMEM((BPW, NNEG), jnp.int32),
          pltpu.VMEM((BPW, DIM), jnp.float32),
          pltpu.VMEM((NBUF, GSZ, DIM), jnp.float32),
          pltpu.VMEM((BPW // 2, NNEG), jnp.float32),
          pltpu.VMEM((BPW,), jnp.float32),
          pltpu.VMEM((4, L), jnp.float32),
          pltpu.VMEM((L, L), jnp.float32),
          pltpu.SemaphoreType.DMA,
          pltpu.SemaphoreType.DMA,
          pltpu.SemaphoreType.DMA,
          pltpu.SemaphoreType.DMA,
      ],
  )
  def sck(ent3_hbm, relidx_hbm, hidx_hbm, tidx_hbm, reltab_hbm,
          pos_out, neg_out, sq_out,
          hidx_v, tidx_v, rele_v, rows_v, scores_v, pos_v,
          sq_v, accv, sem0, sem1, sem2, semg):
    wid = lax.axis_index("s") * NC + lax.axis_index("c")
    base = wid * BPW
    lanes = lax.iota(jnp.int32, L)
    zero = jnp.zeros((L,), jnp.float32)
    cols = [jnp.full((L,), c, jnp.int32) for c in range(L)]
    sems = (sem0, sem1, sem2)

    ent_hbm = ent3_hbm.reshape(B * 2, DIM)

    pltpu.sync_copy(relidx_hbm.at[pl.ds(base, NNEG)], hidx_v.at[0])
    pltpu.sync_copy(relidx_hbm.at[pl.ds(base + NNEG, NNEG)], hidx_v.at[1])
    pltpu.async_copy(reltab_hbm.at[hidx_v.at[0]],
                     rele_v.at[pl.ds(0, NNEG)], semg)
    pltpu.async_copy(reltab_hbm.at[hidx_v.at[1]],
                     rele_v.at[pl.ds(NNEG, NNEG)], semg)
    pltpu.make_async_copy(reltab_hbm.at[hidx_v.at[0]],
                          rele_v.at[pl.ds(0, NNEG)], semg).wait()
    pltpu.make_async_copy(reltab_hbm.at[hidx_v.at[1]],
                          rele_v.at[pl.ds(NNEG, NNEG)], semg).wait()

    pltpu.sync_copy(hidx_hbm.at[pl.ds(base, BPW)], hidx_v)
    pltpu.sync_copy(tidx_hbm.at[pl.ds(base, BPW)], tidx_v)

    def start_gather(b, q, sem):
      pltpu.async_copy(ent_hbm.at[hidx_v.at[b]],
                       rows_v.at[q, pl.ds(0, NNEG)], sem)
      pltpu.async_copy(ent_hbm.at[tidx_v.at[b]],
                       rows_v.at[q, pl.ds(NNEG, NNEG)], sem)

    def wait_gather(b, q, sem):
      pltpu.make_async_copy(ent_hbm.at[hidx_v.at[b]],
                            rows_v.at[q, pl.ds(0, NNEG)], sem).wait()
      pltpu.make_async_copy(ent_hbm.at[tidx_v.at[b]],
                            rows_v.at[q, pl.ds(NNEG, NNEG)], sem).wait()

    start_gather(0, 0, sem0)
    start_gather(1, 1, sem1)

    def load_col(c):
      return plsc.load_gather(accv, [lanes, cols[c]])

    def colsum():
      svec = load_col(0)
      for c in range(1, L):
        svec = svec + load_col(c)
      return svec

    carry = (zero, zero, zero)
    for half in range(2):
      pltpu.sync_copy(ent_hbm.at[pl.ds(base * 2 + half * GSZ, GSZ)],
                      rows_v.at[2])

      def pos_body(g, carry, half=half):
        sqh, sqt, sqr = carry
        for jj in range(L):
          bl = g * L + jj
          b = half * 64 + bl
          acc = zero
          for c in range(NCH):
            lo = c * 16
            hi = HALF + c * 16
            rr = rele_v[b, pl.ds(lo, 16)]
            ri = rele_v[b, pl.ds(hi, 16)]
            hre = rows_v[2, 2 * bl, pl.ds(lo, 16)]
            him = rows_v[2, 2 * bl, pl.ds(hi, 16)]
            tre = rows_v[2, 2 * bl + 1, pl.ds(lo, 16)]
            tim = rows_v[2, 2 * bl + 1, pl.ds(hi, 16)]
            acc = acc + rr * (hre * tre + him * tim) \
                      + ri * (hre * tim - him * tre)
            sqh = sqh + hre * hre + him * him
            sqt = sqt + tre * tre + tim * tim
            sqr = sqr + rr * rr + ri * ri
          accv[jj, :] = acc
        pos_v[pl.ds(half * 64 + g * L, L)] = colsum()
        return sqh, sqt, sqr

      carry = lax.fori_loop(0, 64 // L, pos_body, carry)

    sqh, sqt, sqr = carry
    sq_v[0, :] = sqh
    sq_v[1, :] = sqt
    sq_v[2, :] = sqr
    sq_v[3, :] = zero

    start_gather(2, 2, sem2)

    def do_b(b, q):
        sem = sems[q]
        wait_gather(b, q, sem)

        rr = [rele_v[b, pl.ds(c * 16, 16)] for c in range(NCH)]
        ri = [rele_v[b, pl.ds(HALF + c * 16, 16)] for c in range(NCH)]

        def grp_body(g, _, q=q, rr=rr, ri=ri, b=b):
          for jj in range(L):
            j = g * L + jj
            acc = zero
            for c in range(NCH):
              lo = c * 16
              hi = HALF + c * 16
              hre = rows_v[q, j, pl.ds(lo, 16)]
              him = rows_v[q, j, pl.ds(hi, 16)]
              tre = rows_v[q, NNEG + j, pl.ds(lo, 16)]
              tim = rows_v[q, NNEG + j, pl.ds(hi, 16)]
              acc = acc + rr[c] * (hre * tre + him * tim) \
                        + ri[c] * (hre * tim - him * tre)
            accv[jj, :] = acc
          scores_v[lax.rem(b, BPW // 2), pl.ds(g * L, L)] = colsum()
          return 0

        lax.fori_loop(0, NNEG // L, grp_body, 0)

        @pl.when(b == BPW // 2 - 1)
        def _():
          pltpu.sync_copy(scores_v, neg_out.at[pl.ds(base, BPW // 2)])

        nxt = b + NBUF

        @pl.when(nxt < BPW)
        def _():
          start_gather(nxt, q, sem)

    def neg_body(i, _):
      for q in range(NBUF):
        b = i * NBUF + q
        do_b(b, q)
      return 0

    lax.fori_loop(0, (BPW - 2) // NBUF, neg_body, 0)
    do_b(BPW - 2, (BPW - 2) % NBUF)
    do_b(BPW - 1, (BPW - 1) % NBUF)

    pltpu.sync_copy(pos_v, pos_out.at[wid])
    pltpu.sync_copy(scores_v, neg_out.at[pl.ds(base + BPW // 2, BPW // 2)])
    pltpu.sync_copy(sq_v, sq_out.at[wid])

  return sck(ent3, rel_idx, hidx, tidx, rel_tab)


def _tc_loss(pos2d, neg2d, sq3d):
  def body(pos_ref, neg_ref, sq_ref, out_ref):
    pos = pos_ref[...]
    neg = neg_ref[...]
    sq = sq_ref[...]

    def sp(x):
      return jnp.maximum(x, 0.0) + jnp.log1p(jnp.exp(-jnp.abs(x)))

    model = 0.5 * (jnp.sum(sp(-pos)) / B + jnp.sum(sp(neg)) / (B * NNEG))
    reg = REG * jnp.sum(sq) / (3.0 * B * DIM)
    out_ref[0, 0] = model + reg

  out = pl.pallas_call(
      body,
      out_shape=jax.ShapeDtypeStruct((1, 1), jnp.float32),
      out_specs=pl.BlockSpec(memory_space=pltpu.SMEM),
  )(pos2d, neg2d, sq3d)
  return out[0, 0]


def kernel(ent_embs, rels, neg_idx, rel_emb_weight):
  rel_idx = rels.reshape(B).astype(jnp.int32)
  hidx = neg_idx[:, :, 0]
  tidx = neg_idx[:, :, 1]
  pos, neg, sq = _sc_scores(ent_embs, rel_idx, hidx, tidx, rel_emb_weight)
  return _tc_loss(pos, neg, sq)
